# Initial kernel scaffold; baseline (speedup 1.0000x reference)
import jax
import jax.numpy as jnp
from jax.experimental import pallas as pl


def kernel(x, edge_index, batch, params):
    raise NotImplementedError("write your pallas kernel here")


# speedup vs baseline: 3.2203x; 3.2203x over previous
import functools

import jax
import jax.numpy as jnp
from jax import lax
from jax.experimental import pallas as pl
from jax.experimental.pallas import tpu as pltpu
from jax.experimental.pallas import tpu_sc as plsc

R = 2
N = 10000
NT = R * N
E = 160000
E2 = R * E
D = 128
OUT = 112
G = 128
NUM_LAYERS = 4

NS = 16
NCORES = 2

CHUNK = 128
MP_CH = 158
EP = NS * MP_CH * CHUNK

HALF = 10048
ACC_ROWS = 11136
TPT = 696
NTRASH = ACC_ROWS - HALF
XT = 21248


@functools.cache
def _sc_kernels():
    mesh = plsc.VectorSubcoreMesh(
        core_axis_name="c", subcore_axis_name="s", num_cores=NCORES, num_subcores=NS
    )

    @functools.partial(
        pl.kernel,
        out_type=jax.ShapeDtypeStruct((NCORES * ACC_ROWS, D), jnp.float32),
        mesh=mesh,
        scratch_types=[
            pltpu.VMEM((1, CHUNK), jnp.int32),
            pltpu.VMEM((1, CHUNK), jnp.int32),
            pltpu.VMEM((CHUNK, D), jnp.float32),
            pltpu.VMEM_SHARED((ACC_ROWS, D), jnp.float32),
        ],
    )
    def mp_kernel(xs_hbm, row_hbm, col_hbm, out_hbm, rows_v, cols_v, gbuf, acc):
        c = lax.axis_index("c")
        s = lax.axis_index("s")
        base = pl.multiple_of(s * TPT, 8)
        ibase = pl.multiple_of(c * HALF + base, 8)
        pltpu.sync_copy(xs_hbm.at[pl.ds(ibase, TPT)], acc.at[pl.ds(base, TPT)])
        plsc.subcore_barrier()

        def body(j, carry):
            pltpu.sync_copy(row_hbm.at[s, j], rows_v.at[0])
            pltpu.sync_copy(col_hbm.at[c, s, j], cols_v.at[0])
            pltpu.sync_copy(xs_hbm.at[rows_v.at[0]], gbuf)
            pltpu.sync_copy(gbuf, acc.at[cols_v.at[0]], add=True)
            return carry

        lax.fori_loop(0, MP_CH, body, 0)
        plsc.subcore_barrier()
        obase = pl.multiple_of(c * ACC_ROWS + base, 8)
        pltpu.sync_copy(acc.at[pl.ds(base, TPT)], out_hbm.at[pl.ds(obase, TPT)])

    return mp_kernel


def _bn(h, g, b):
    mu = jnp.mean(h, axis=0, keepdims=True)
    va = jnp.mean((h - mu) * (h - mu), axis=0, keepdims=True)
    return (h - mu) / jnp.sqrt(va + 1e-5) * g + b


def _dot(a, b):
    return jnp.dot(a, b, preferred_element_type=jnp.float32)


def _conv_stage(xf, dinvb, W1t, b1, g1, be1, W2t, b2, Wgt):
    h = _dot(xf, W1t) + b1
    h = jnp.maximum(_bn(h, g1, be1), 0.0)
    h = _dot(h, W2t) + b2
    xt = _dot(h, Wgt)
    return xt * dinvb


def _write_xs(xs_out, xs):
    xs_out[0:NT, :] = xs
    xs_out[NT:XT, :] = jnp.zeros((XT - NT, D), jnp.float32)


def _pool_y(P, m, fWt, fb):
    pooled = lax.dot_general(P, m, (((0,), (0,)), ((), ())), preferred_element_type=jnp.float32)
    return _dot(pooled, fWt) + fb


def _tc0_body(xf_ref, xm_ref, dinvb_ref, P_ref, W1t, b1, g1, be1, W2t, b2, Wgt, fWt, fb,
              xs_out, y_out):
    xs = _conv_stage(xf_ref[...], dinvb_ref[...], W1t[...], b1[...], g1[...], be1[...],
                     W2t[...], b2[...], Wgt[...])
    _write_xs(xs_out, xs)
    y_out[...] = _pool_y(P_ref[...], xm_ref[...], fWt[...], fb[...])


def _consume(agg_ref, dinvb, bgb, bng, bnb):
    agg = jnp.concatenate(
        [agg_ref[0:HALF, :], agg_ref[ACC_ROWS : ACC_ROWS + (NT - HALF), :]], axis=0)
    ht = agg * dinvb + bgb
    return jnp.maximum(_bn(ht, bng, bnb), 0.0)


def _tc_mid_body(agg_ref, dinvb_ref, P_ref, bgb, bng, bnb, W1t, b1, g1, be1, W2t, b2, Wgt,
                 fWt, fb, xs_out, y_out):
    dinvb = dinvb_ref[...]
    hn = _consume(agg_ref, dinvb, bgb[...], bng[...], bnb[...])
    m = 0.5 * (hn[0:N, :] + hn[N:NT, :])
    y_out[...] = _pool_y(P_ref[...], m, fWt[...], fb[...])
    xs = _conv_stage(hn, dinvb, W1t[...], b1[...], g1[...], be1[...], W2t[...], b2[...],
                     Wgt[...])
    _write_xs(xs_out, xs)


def _tc_final_body(agg_ref, dinvb_ref, P_ref, bgb, bng, bnb, fWt, fb, y_out):
    hn = _consume(agg_ref, dinvb_ref[...], bgb[...], bng[...], bnb[...])
    m = 0.5 * (hn[0:N, :] + hn[N:NT, :])
    y_out[...] = _pool_y(P_ref[...], m, fWt[...], fb[...])


_XS_TYPE = jax.ShapeDtypeStruct((XT, D), jnp.float32)
_Y_TYPE = jax.ShapeDtypeStruct((G, OUT), jnp.float32)

_tc0 = pl.pallas_call(_tc0_body, out_shape=[_XS_TYPE, _Y_TYPE])
_tc_mid = pl.pallas_call(_tc_mid_body, out_shape=[_XS_TYPE, _Y_TYPE])
_tc_final = pl.pallas_call(_tc_final_body, out_shape=_Y_TYPE)


def kernel(x, edge_index, batch, params):
    f32 = jnp.float32
    drop = jax.random.bernoulli(jax.random.key(42), 0.2, (R, N))
    scale = 1.0 - drop.astype(f32)
    xf0 = jnp.concatenate([x * scale[0][:, None], x * scale[1][:, None]], axis=0)
    xm0 = x * ((scale[0] + scale[1]) * 0.5)[:, None]

    em = jnp.max(edge_index) + 1
    row = jnp.concatenate([edge_index[0], edge_index[0] + em])
    col = jnp.concatenate([edge_index[1], edge_index[1] + em])
    pad = EP - E2
    rowp = jnp.concatenate([row, jnp.zeros((pad,), jnp.int32)])
    colp = jnp.concatenate([col, jnp.full((pad,), NT, jnp.int32)])
    trash = HALF + (jnp.arange(EP, dtype=jnp.int32) % NTRASH)
    lcol0 = jnp.where(colp < HALF, colp, trash)
    lcol1 = jnp.where((colp >= HALF) & (colp < NT), colp - HALF, trash)
    row_mp = rowp.reshape(NS, MP_CH, CHUNK)
    col_mp = jnp.stack([lcol0, lcol1]).reshape(NCORES, NS, MP_CH, CHUNK)

    mp_kernel = _sc_kernels()

    ones_tab = jnp.ones((XT, D), f32)
    deg_cat = mp_kernel(ones_tab, row_mp, col_mp)
    deg = jnp.concatenate(
        [deg_cat[0:HALF, 0], deg_cat[ACC_ROWS : ACC_ROWS + (NT - HALF), 0]], axis=0)
    dinv = jnp.where(deg > 0, lax.rsqrt(deg), 0.0)
    dinvb = jnp.broadcast_to(dinv[:, None], (NT, D))

    P = (batch[:, None] == jnp.arange(G, dtype=batch.dtype)[None, :]).astype(f32)

    def r2(v):
        return v.reshape(1, -1)

    convs = params["convs"]
    fcs = params["fcs"]
    bns = params["bns"]

    c0 = convs[0]
    xs, y = _tc0(
        xf0, xm0, dinvb, P,
        c0["W1"].T, r2(c0["b1"]), r2(c0["g1"]), r2(c0["be1"]),
        c0["W2"].T, r2(c0["b2"]), c0["Wg"].T,
        fcs[0]["W"].T, r2(fcs[0]["b"]),
    )
    out = y
    for i in range(NUM_LAYERS):
        agg = mp_kernel(xs, row_mp, col_mp)
        bgb = r2(convs[i]["bg"])
        bng, bnb = r2(bns[i]["g"]), r2(bns[i]["b"])
        fWt, fb = fcs[i + 1]["W"].T, r2(fcs[i + 1]["b"])
        if i < NUM_LAYERS - 1:
            cn = convs[i + 1]
            xs, y = _tc_mid(
                agg, dinvb, P, bgb, bng, bnb,
                cn["W1"].T, r2(cn["b1"]), r2(cn["g1"]), r2(cn["be1"]),
                cn["W2"].T, r2(cn["b2"]), cn["Wg"].T,
                fWt, fb,
            )
        else:
            y = _tc_final(agg, dinvb, P, bgb, bng, bnb, fWt, fb)
        out = out + y
    return out


# speedup vs baseline: 4.2794x; 1.3289x over previous
import functools

import jax
import jax.numpy as jnp
from jax import lax
from jax.experimental import pallas as pl
from jax.experimental.pallas import tpu as pltpu
from jax.experimental.pallas import tpu_sc as plsc

R = 2
N = 10000
NT = R * N
E = 160000
E2 = R * E
D = 128
OUT = 112
G = 128
NUM_LAYERS = 4

NS = 16
NCORES = 2

CHUNK = 128
MP_CH = 158
EP = NS * MP_CH * CHUNK

HALF = 10048
ACC_ROWS = 11136
TPT = 696
NTRASH = ACC_ROWS - HALF
XT = 21248


@functools.cache
def _sc_kernels():
    mesh = plsc.VectorSubcoreMesh(
        core_axis_name="c", subcore_axis_name="s", num_cores=NCORES, num_subcores=NS
    )

    @functools.partial(
        pl.kernel,
        out_type=jax.ShapeDtypeStruct((NCORES * ACC_ROWS, D), jnp.float32),
        mesh=mesh,
        scratch_types=[
            pltpu.VMEM((2, 2, CHUNK), jnp.int32),
            pltpu.VMEM((2, CHUNK, D), jnp.float32),
            pltpu.VMEM_SHARED((ACC_ROWS, D), jnp.float32),
            pltpu.SemaphoreType.DMA,
            pltpu.SemaphoreType.DMA,
            pltpu.SemaphoreType.DMA,
            pltpu.SemaphoreType.DMA,
        ],
    )
    def mp_kernel(xs_hbm, idx_hbm, out_hbm, idxb, gbuf, acc, semg0, semg1, sems0, sems1):
        c = lax.axis_index("c")
        s = lax.axis_index("s")
        semg = (semg0, semg1)
        sems = (sems0, sems1)

        def idxload(j, slot):
            pltpu.sync_copy(idx_hbm.at[c, s, j], idxb.at[slot])

        def gather_start(slot):
            return pltpu.async_copy(
                xs_hbm.at[idxb.at[slot, 0]], gbuf.at[slot], semg[slot])

        def scatter_start(slot):
            return pltpu.async_copy(
                gbuf.at[slot], acc.at[idxb.at[slot, 1]], sems[slot], add=True)

        def gather_wait(slot):
            pltpu.make_async_copy(
                xs_hbm.at[idxb.at[slot, 0]], gbuf.at[slot], semg[slot]).wait()

        idxload(0, 0)
        idxload(1, 1)
        gather_start(0)
        gather_start(1)
        base = pl.multiple_of(s * TPT, 8)
        ibase = pl.multiple_of(c * HALF + base, 8)
        pltpu.sync_copy(xs_hbm.at[pl.ds(ibase, TPT)], acc.at[pl.ds(base, TPT)])
        plsc.subcore_barrier()

        npairs = MP_CH // 2

        def body(k, carry):
            gather_wait(0)
            sc0 = scatter_start(0)
            gather_wait(1)
            sc1 = scatter_start(1)

            @pl.when(k < npairs - 1)
            def _():
                sc0.wait()
                idxload(2 * k + 2, 0)
                gather_start(0)
                sc1.wait()
                idxload(2 * k + 3, 1)
                gather_start(1)

            @pl.when(k == npairs - 1)
            def _():
                sc0.wait()
                sc1.wait()

            return carry

        lax.fori_loop(0, npairs, body, 0)
        plsc.subcore_barrier()
        obase = pl.multiple_of(c * ACC_ROWS + base, 8)
        pltpu.sync_copy(acc.at[pl.ds(base, TPT)], out_hbm.at[pl.ds(obase, TPT)])

    return mp_kernel


def _bn(h, g, b):
    mu = jnp.mean(h, axis=0, keepdims=True)
    va = jnp.mean((h - mu) * (h - mu), axis=0, keepdims=True)
    return (h - mu) / jnp.sqrt(va + 1e-5) * g + b


def _dot(a, b):
    return jnp.dot(a, b, preferred_element_type=jnp.float32)


def _conv_stage(xf, dinvb, W1t, b1, g1, be1, W2t, b2, Wgt):
    h = _dot(xf, W1t) + b1
    h = jnp.maximum(_bn(h, g1, be1), 0.0)
    h = _dot(h, W2t) + b2
    xt = _dot(h, Wgt)
    return xt * dinvb


def _write_xs(xs_out, xs):
    xs_out[0:NT, :] = xs
    xs_out[NT:XT, :] = jnp.zeros((XT - NT, D), jnp.float32)


def _pool_y(P, m, fWt, fb):
    pooled = lax.dot_general(P, m, (((0,), (0,)), ((), ())), preferred_element_type=jnp.float32)
    return _dot(pooled, fWt) + fb


def _tc0_body(xf_ref, xm_ref, dinvb_ref, P_ref, W1t, b1, g1, be1, W2t, b2, Wgt, fWt, fb,
              xs_out, y_out):
    xs = _conv_stage(xf_ref[...], dinvb_ref[...], W1t[...], b1[...], g1[...], be1[...],
                     W2t[...], b2[...], Wgt[...])
    _write_xs(xs_out, xs)
    y_out[...] = _pool_y(P_ref[...], xm_ref[...], fWt[...], fb[...])


def _consume(agg_ref, dinvb, bgb, bng, bnb):
    agg = jnp.concatenate(
        [agg_ref[0:HALF, :], agg_ref[ACC_ROWS : ACC_ROWS + (NT - HALF), :]], axis=0)
    ht = agg * dinvb + bgb
    return jnp.maximum(_bn(ht, bng, bnb), 0.0)


def _tc_mid_body(agg_ref, dinvb_ref, P_ref, bgb, bng, bnb, W1t, b1, g1, be1, W2t, b2, Wgt,
                 fWt, fb, xs_out, y_out):
    dinvb = dinvb_ref[...]
    hn = _consume(agg_ref, dinvb, bgb[...], bng[...], bnb[...])
    m = 0.5 * (hn[0:N, :] + hn[N:NT, :])
    y_out[...] = _pool_y(P_ref[...], m, fWt[...], fb[...])
    xs = _conv_stage(hn, dinvb, W1t[...], b1[...], g1[...], be1[...], W2t[...], b2[...],
                     Wgt[...])
    _write_xs(xs_out, xs)


def _tc_final_body(agg_ref, dinvb_ref, P_ref, bgb, bng, bnb, fWt, fb, y_out):
    hn = _consume(agg_ref, dinvb_ref[...], bgb[...], bng[...], bnb[...])
    m = 0.5 * (hn[0:N, :] + hn[N:NT, :])
    y_out[...] = _pool_y(P_ref[...], m, fWt[...], fb[...])


_XS_TYPE = jax.ShapeDtypeStruct((XT, D), jnp.float32)
_Y_TYPE = jax.ShapeDtypeStruct((G, OUT), jnp.float32)

_tc0 = pl.pallas_call(_tc0_body, out_shape=[_XS_TYPE, _Y_TYPE])
_tc_mid = pl.pallas_call(_tc_mid_body, out_shape=[_XS_TYPE, _Y_TYPE])
_tc_final = pl.pallas_call(_tc_final_body, out_shape=_Y_TYPE)


def kernel(x, edge_index, batch, params):
    f32 = jnp.float32
    drop = jax.random.bernoulli(jax.random.key(42), 0.2, (R, N))
    scale = 1.0 - drop.astype(f32)
    xf0 = jnp.concatenate([x * scale[0][:, None], x * scale[1][:, None]], axis=0)
    xm0 = x * ((scale[0] + scale[1]) * 0.5)[:, None]

    em = jnp.max(edge_index) + 1
    row = jnp.concatenate([edge_index[0], edge_index[0] + em])
    col = jnp.concatenate([edge_index[1], edge_index[1] + em])
    pad = EP - E2
    rowp = jnp.concatenate([row, jnp.zeros((pad,), jnp.int32)])
    colp = jnp.concatenate([col, jnp.full((pad,), NT, jnp.int32)])
    trash = HALF + (jnp.arange(EP, dtype=jnp.int32) % NTRASH)
    lcol0 = jnp.where(colp < HALF, colp, trash)
    lcol1 = jnp.where((colp >= HALF) & (colp < NT), colp - HALF, trash)
    rows2 = jnp.broadcast_to(rowp.reshape(1, NS, MP_CH, CHUNK), (NCORES, NS, MP_CH, CHUNK))
    cols2 = jnp.stack([lcol0, lcol1]).reshape(NCORES, NS, MP_CH, CHUNK)
    idx_mp = jnp.stack([rows2, cols2], axis=3)

    mp_kernel = _sc_kernels()

    ones_tab = jnp.ones((XT, D), f32)
    deg_cat = mp_kernel(ones_tab, idx_mp)
    deg = jnp.concatenate(
        [deg_cat[0:HALF, 0], deg_cat[ACC_ROWS : ACC_ROWS + (NT - HALF), 0]], axis=0)
    dinv = jnp.where(deg > 0, lax.rsqrt(deg), 0.0)
    dinvb = jnp.broadcast_to(dinv[:, None], (NT, D))

    P = (batch[:, None] == jnp.arange(G, dtype=batch.dtype)[None, :]).astype(f32)

    def r2(v):
        return v.reshape(1, -1)

    convs = params["convs"]
    fcs = params["fcs"]
    bns = params["bns"]

    c0 = convs[0]
    xs, y = _tc0(
        xf0, xm0, dinvb, P,
        c0["W1"].T, r2(c0["b1"]), r2(c0["g1"]), r2(c0["be1"]),
        c0["W2"].T, r2(c0["b2"]), c0["Wg"].T,
        fcs[0]["W"].T, r2(fcs[0]["b"]),
    )
    out = y
    for i in range(NUM_LAYERS):
        agg = mp_kernel(xs, idx_mp)
        bgb = r2(convs[i]["bg"])
        bng, bnb = r2(bns[i]["g"]), r2(bns[i]["b"])
        fWt, fb = fcs[i + 1]["W"].T, r2(fcs[i + 1]["b"])
        if i < NUM_LAYERS - 1:
            cn = convs[i + 1]
            xs, y = _tc_mid(
                agg, dinvb, P, bgb, bng, bnb,
                cn["W1"].T, r2(cn["b1"]), r2(cn["g1"]), r2(cn["be1"]),
                cn["W2"].T, r2(cn["b2"]), cn["Wg"].T,
                fWt, fb,
            )
        else:
            y = _tc_final(agg, dinvb, P, bgb, bng, bnb, fWt, fb)
        out = out + y
    return out
